# baseline (speedup 1.0000x reference)
import jax
import jax.numpy as jnp
from jax import lax
from jax.experimental import pallas as pl

_N = 512


def _sort_tc_body(xr_ref, mr_ref, idx_ref, vals_ref):
    xr = xr_ref[...]
    mr = mr_ref[...]

    jc = lax.broadcasted_iota(jnp.int32, (_N, _N), 1)
    ic = lax.broadcasted_iota(jnp.int32, (_N, _N), 0)
    diag = jc == ic

    xc = jnp.sum(jnp.where(diag, xr, 0.0), axis=1, keepdims=True)
    mc = jnp.sum(jnp.where(diag, mr, 0), axis=1, keepdims=True)

    ahead = (xr > xc) | ((xr == xc) & (jc < ic))
    rank = jnp.sum(ahead.astype(jnp.int32), axis=1, keepdims=True)

    onehot = rank == jc
    vals_ref[...] = jnp.sum(jnp.where(onehot, xc, 0.0), axis=0, keepdims=True)
    idx_ref[...] = jnp.sum(jnp.where(onehot, mc, 0), axis=0, keepdims=True)


@jax.jit
def kernel(input, map_arr):
    xr = input.reshape(1, _N)
    mr = map_arr.reshape(1, _N)
    out_idx, out_vals = pl.pallas_call(
        _sort_tc_body,
        out_shape=(
            jax.ShapeDtypeStruct((1, _N), jnp.int32),
            jax.ShapeDtypeStruct((1, _N), jnp.float32),
        ),
    )(xr, mr)
    return out_idx.reshape(_N), out_vals.reshape(_N)
